# baseline (speedup 1.0000x reference)
import functools

import jax
import jax.numpy as jnp
from jax.experimental import pallas as pl
from jax.experimental.pallas import tpu as pltpu

_F32 = jnp.float32
_BF16 = jnp.bfloat16
_I32 = jnp.int32


def _gelu_exact(x):
    return 0.5 * x * (1.0 + jax.lax.erf(x * 0.7071067811865476))


def _gather_body(hl_ref, mc_ref, uc_ref, hu_ref, hm_ref, *, ST, U, M):
    s = pl.program_id(1)
    base = s * ST

    @pl.when(s == 0)
    def _z():
        hu_ref[...] = jnp.zeros_like(hu_ref)
        hm_ref[...] = jnp.zeros_like(hm_ref)

    hl = hl_ref[0]
    gu = (jax.lax.broadcasted_iota(_I32, (U, ST), 1) + base
          == uc_ref[0]).astype(_BF16)
    hu_ref[0] += jnp.dot(gu, hl, preferred_element_type=_F32).astype(_BF16)
    gm = (jax.lax.broadcasted_iota(_I32, (M, ST), 1) + base
          == mc_ref[0]).astype(_BF16)
    hm_ref[0] += jnp.dot(gm, hl, preferred_element_type=_F32)


def _moe_body(hu_ref, hm_ref, mc_ref, ur_ref, r_ref, wr_ref, br_ref,
              w1_ref, b1_ref, w2_ref, b2_ref, ln_ref,
              w_s, adjn_s, cpos_s, mix_s, *, K):
    k = pl.program_id(0)
    b = pl.program_id(1)

    @pl.when(k == 0)
    def _init():
        logits = jnp.dot(hm_ref[b], wr_ref[...], preferred_element_type=_F32)
        logits = logits + br_ref[...]
        mx = jnp.max(logits, axis=-1, keepdims=True)
        e = jnp.exp(logits - mx)
        w_s[b] = e / jnp.sum(e, axis=-1, keepdims=True)
        diff = jnp.abs(ur_ref[b] - mc_ref[b])
        adj = ((diff > 0) & (diff <= r_ref[0])).astype(_F32)
        cnt = jnp.sum(adj, axis=-1, keepdims=True)
        adjn_s[b] = adj / jnp.maximum(cnt, 1.0)
        cpos_s[b] = (cnt > 0.0).astype(_F32)
        mix_s[b] = jnp.dot(w_s[b], b2_ref[...], preferred_element_type=_F32)

    hid = jnp.dot(hu_ref[b], w1_ref[0], preferred_element_type=_F32)
    hid = _gelu_exact(hid + b1_ref[0]).astype(_BF16)
    sel = (jax.lax.broadcasted_iota(_I32, (1, K), 1) == k).astype(_F32)
    w_col = jnp.sum(w_s[b] * sel, axis=-1, keepdims=True)
    aw = (adjn_s[b] * w_col).astype(_BF16)
    t = jnp.dot(aw, hid, preferred_element_type=_F32).astype(_BF16)
    mix_s[b] += jnp.dot(t, w2_ref[0], preferred_element_type=_F32)

    @pl.when(k == K - 1)
    def _fin():
        mix = mix_s[b]
        mu = jnp.mean(mix, axis=-1, keepdims=True)
        var = jnp.mean((mix - mu) ** 2, axis=-1, keepdims=True)
        ln_ref[b] = (mix - mu) * jax.lax.rsqrt(var + 1e-5) * cpos_s[b]


def _scatter_body(mr_ref, ln_ref, out_ref, *, ST, M):
    s = pl.program_id(1)
    base = s * ST
    mr = mr_ref[0]
    nxt = jnp.concatenate([mr[:, 1:], jnp.full((1, 1), -1, _I32)], axis=1)
    last = mr != nxt
    col = jax.lax.broadcasted_iota(_I32, (ST, M), 0) + base
    pm = ((col == mr) & last).astype(_F32)
    out_ref[0] = jnp.dot(pm, ln_ref[0], preferred_element_type=_F32)


def kernel(h_L, mask_indices, unmasked_indices, range_r, W_r, b_r,
           W1, b1, W2, b2):
    B, S, D = h_L.shape
    M = mask_indices.shape[1]
    U = unmasked_indices.shape[1]
    K = W_r.shape[1]
    F = W1.shape[2]
    ST = 512
    NS = S // ST
    mi = mask_indices.astype(_I32)
    ui = unmasked_indices.astype(_I32)
    r_arr = jnp.asarray(range_r, _I32).reshape(1)

    hu, hm = pl.pallas_call(
        functools.partial(_gather_body, ST=ST, U=U, M=M),
        grid=(B, NS),
        in_specs=[
            pl.BlockSpec((1, ST, D), lambda b, s: (b, s, 0)),
            pl.BlockSpec((1, M, 1), lambda b, s: (b, 0, 0)),
            pl.BlockSpec((1, U, 1), lambda b, s: (b, 0, 0)),
        ],
        out_specs=[
            pl.BlockSpec((1, U, D), lambda b, s: (b, 0, 0)),
            pl.BlockSpec((1, M, D), lambda b, s: (b, 0, 0)),
        ],
        out_shape=[
            jax.ShapeDtypeStruct((B, U, D), _BF16),
            jax.ShapeDtypeStruct((B, M, D), _F32),
        ],
    )(h_L.astype(_BF16), mi.reshape(B, M, 1), ui.reshape(B, U, 1))

    ln = pl.pallas_call(
        functools.partial(_moe_body, K=K),
        grid=(K, B),
        in_specs=[
            pl.BlockSpec((B, U, D), lambda k, b: (0, 0, 0)),
            pl.BlockSpec((B, M, D), lambda k, b: (0, 0, 0)),
            pl.BlockSpec((B, M, 1), lambda k, b: (0, 0, 0)),
            pl.BlockSpec((B, 1, U), lambda k, b: (0, 0, 0)),
            pl.BlockSpec(memory_space=pltpu.SMEM),
            pl.BlockSpec((D, K), lambda k, b: (0, 0)),
            pl.BlockSpec((1, K), lambda k, b: (0, 0)),
            pl.BlockSpec((1, D, F), lambda k, b: (k, 0, 0)),
            pl.BlockSpec((1, 1, F), lambda k, b: (k, 0, 0)),
            pl.BlockSpec((1, F, D), lambda k, b: (k, 0, 0)),
            pl.BlockSpec((K, D), lambda k, b: (0, 0)),
        ],
        out_specs=pl.BlockSpec((B, M, D), lambda k, b: (0, 0, 0)),
        out_shape=jax.ShapeDtypeStruct((B, M, D), _F32),
        scratch_shapes=[
            pltpu.VMEM((B, M, K), _F32),
            pltpu.VMEM((B, M, U), _F32),
            pltpu.VMEM((B, M, 1), _F32),
            pltpu.VMEM((B, M, D), _F32),
        ],
    )(hu, hm, mi.reshape(B, M, 1), ui.reshape(B, 1, U), r_arr,
      W_r, b_r.reshape(1, K), W1.astype(_BF16), b1.reshape(K, 1, F),
      W2.astype(_BF16), b2)

    out = pl.pallas_call(
        functools.partial(_scatter_body, ST=ST, M=M),
        grid=(B, NS),
        in_specs=[
            pl.BlockSpec((1, 1, M), lambda b, s: (b, 0, 0)),
            pl.BlockSpec((1, M, D), lambda b, s: (b, 0, 0)),
        ],
        out_specs=pl.BlockSpec((1, ST, D), lambda b, s: (b, s, 0)),
        out_shape=jax.ShapeDtypeStruct((B, S, D), _F32),
    )(mi.reshape(B, 1, M), ln)
    return out


# speedup vs baseline: 1.4590x; 1.4590x over previous
import functools

import jax
import jax.numpy as jnp
from jax.experimental import pallas as pl
from jax.experimental.pallas import tpu as pltpu

_F32 = jnp.float32
_BF16 = jnp.bfloat16
_I32 = jnp.int32


def _gelu_exact(x):
    return 0.5 * x * (1.0 + jax.lax.erf(x * 0.7071067811865476))


def _gather_body(hl_ref, mc_ref, uc_ref, hu_ref, hm_ref, *, ST, U, M):
    s = pl.program_id(1)
    base = s * ST

    @pl.when(s == 0)
    def _z():
        hu_ref[...] = jnp.zeros_like(hu_ref)
        hm_ref[...] = jnp.zeros_like(hm_ref)

    hl = hl_ref[0]
    gu = (jax.lax.broadcasted_iota(_I32, (U, ST), 1) + base
          == uc_ref[0]).astype(_F32)
    hu_ref[0] += jnp.dot(gu, hl, preferred_element_type=_F32).astype(_BF16)
    gm = (jax.lax.broadcasted_iota(_I32, (M, ST), 1) + base
          == mc_ref[0]).astype(_F32)
    hm_ref[0] += jnp.dot(gm, hl, preferred_element_type=_F32)


def _moe_body(hu_ref, hm_ref, mc_ref, ur_ref, r_ref, wr_ref, br_ref,
              w1_ref, b1_ref, w2_ref, b2_ref, ln_ref,
              w_s, adjn_s, cpos_s, mix_s, *, K):
    k = pl.program_id(0)
    b = pl.program_id(1)

    @pl.when(k == 0)
    def _init():
        logits = jnp.dot(hm_ref[b], wr_ref[...], preferred_element_type=_F32)
        logits = logits + br_ref[...]
        mx = jnp.max(logits, axis=-1, keepdims=True)
        e = jnp.exp(logits - mx)
        w_s[b] = e / jnp.sum(e, axis=-1, keepdims=True)
        diff = jnp.abs(ur_ref[b] - mc_ref[b])
        adj = ((diff > 0) & (diff <= r_ref[0])).astype(_F32)
        cnt = jnp.sum(adj, axis=-1, keepdims=True)
        adjn_s[b] = adj / jnp.maximum(cnt, 1.0)
        cpos_s[b] = (cnt > 0.0).astype(_F32)
        mix_s[b] = jnp.dot(w_s[b], b2_ref[...], preferred_element_type=_F32)

    hid = jnp.dot(hu_ref[b], w1_ref[0].astype(_BF16),
                  preferred_element_type=_F32)
    hid = _gelu_exact(hid + b1_ref[0]).astype(_BF16)
    sel = (jax.lax.broadcasted_iota(_I32, (1, K), 1) == k).astype(_F32)
    w_col = jnp.sum(w_s[b] * sel, axis=-1, keepdims=True)
    aw = (adjn_s[b] * w_col).astype(_BF16)
    t = jnp.dot(aw, hid, preferred_element_type=_F32).astype(_BF16)
    mix_s[b] += jnp.dot(t, w2_ref[0].astype(_BF16),
                        preferred_element_type=_F32)

    @pl.when(k == K - 1)
    def _fin():
        mix = mix_s[b]
        mu = jnp.mean(mix, axis=-1, keepdims=True)
        var = jnp.mean((mix - mu) ** 2, axis=-1, keepdims=True)
        ln_ref[b] = (mix - mu) * jax.lax.rsqrt(var + 1e-5) * cpos_s[b]


def _scatter_body(mr_ref, ln_ref, out_ref, *, ST, M):
    s = pl.program_id(1)
    base = s * ST
    mr = mr_ref[0]
    nxt = jnp.concatenate([mr[:, 1:], jnp.full((1, 1), -1, _I32)], axis=1)
    last = mr != nxt
    col = jax.lax.broadcasted_iota(_I32, (ST, M), 0) + base
    pm = ((col == mr) & last).astype(_F32)
    out_ref[0] = jnp.dot(pm, ln_ref[0], preferred_element_type=_F32)


def kernel(h_L, mask_indices, unmasked_indices, range_r, W_r, b_r,
           W1, b1, W2, b2):
    B, S, D = h_L.shape
    M = mask_indices.shape[1]
    U = unmasked_indices.shape[1]
    K = W_r.shape[1]
    F = W1.shape[2]
    ST = 512
    NS = S // ST
    mi = mask_indices.astype(_I32)
    ui = unmasked_indices.astype(_I32)
    r_arr = jnp.asarray(range_r, _I32).reshape(1)

    hu, hm = pl.pallas_call(
        functools.partial(_gather_body, ST=ST, U=U, M=M),
        grid=(B, NS),
        in_specs=[
            pl.BlockSpec((1, ST, D), lambda b, s: (b, s, 0)),
            pl.BlockSpec((1, M, 1), lambda b, s: (b, 0, 0)),
            pl.BlockSpec((1, U, 1), lambda b, s: (b, 0, 0)),
        ],
        out_specs=[
            pl.BlockSpec((1, U, D), lambda b, s: (b, 0, 0)),
            pl.BlockSpec((1, M, D), lambda b, s: (b, 0, 0)),
        ],
        out_shape=[
            jax.ShapeDtypeStruct((B, U, D), _BF16),
            jax.ShapeDtypeStruct((B, M, D), _F32),
        ],
    )(h_L, mi.reshape(B, M, 1), ui.reshape(B, U, 1))

    ln = pl.pallas_call(
        functools.partial(_moe_body, K=K),
        grid=(K, B),
        in_specs=[
            pl.BlockSpec((B, U, D), lambda k, b: (0, 0, 0)),
            pl.BlockSpec((B, M, D), lambda k, b: (0, 0, 0)),
            pl.BlockSpec((B, M, 1), lambda k, b: (0, 0, 0)),
            pl.BlockSpec((B, 1, U), lambda k, b: (0, 0, 0)),
            pl.BlockSpec(memory_space=pltpu.SMEM),
            pl.BlockSpec((D, K), lambda k, b: (0, 0)),
            pl.BlockSpec((1, K), lambda k, b: (0, 0)),
            pl.BlockSpec((1, D, F), lambda k, b: (k, 0, 0)),
            pl.BlockSpec((1, 1, F), lambda k, b: (k, 0, 0)),
            pl.BlockSpec((1, F, D), lambda k, b: (k, 0, 0)),
            pl.BlockSpec((K, D), lambda k, b: (0, 0)),
        ],
        out_specs=pl.BlockSpec((B, M, D), lambda k, b: (0, 0, 0)),
        out_shape=jax.ShapeDtypeStruct((B, M, D), _F32),
        scratch_shapes=[
            pltpu.VMEM((B, M, K), _F32),
            pltpu.VMEM((B, M, U), _F32),
            pltpu.VMEM((B, M, 1), _F32),
            pltpu.VMEM((B, M, D), _F32),
        ],
    )(hu, hm, mi.reshape(B, M, 1), ui.reshape(B, 1, U), r_arr,
      W_r, b_r.reshape(1, K), W1, b1.reshape(K, 1, F), W2, b2)

    out = pl.pallas_call(
        functools.partial(_scatter_body, ST=ST, M=M),
        grid=(B, NS),
        in_specs=[
            pl.BlockSpec((1, 1, M), lambda b, s: (b, 0, 0)),
            pl.BlockSpec((1, M, D), lambda b, s: (b, 0, 0)),
        ],
        out_specs=pl.BlockSpec((1, ST, D), lambda b, s: (b, s, 0)),
        out_shape=jax.ShapeDtypeStruct((B, S, D), _F32),
    )(mi.reshape(B, 1, M), ln)
    return out


# speedup vs baseline: 1.8249x; 1.2508x over previous
import functools

import jax
import jax.numpy as jnp
from jax.experimental import pallas as pl
from jax.experimental.pallas import tpu as pltpu

_F32 = jnp.float32
_BF16 = jnp.bfloat16
_I32 = jnp.int32


def _gelu_exact(x):
    return 0.5 * x * (1.0 + jax.lax.erf(x * 0.7071067811865476))


def _body(hl_ref, mc_ref, mr_ref, uc_ref, ur_ref, r_ref, wr_ref, br_ref,
          w1_ref, b1_ref, w2_ref, b2_ref, out_ref,
          hu_s, hm_s, w_s, adjn_s, cpos_s, mix_s,
          *, B, NS, ST, U, M, K, D):
    p = pl.program_id(0)
    NSB = B * NS
    BM = B * M
    BU = B * U

    @pl.when(p == 0)
    def _zero():
        hu_s[...] = jnp.zeros_like(hu_s)
        hm_s[...] = jnp.zeros_like(hm_s)

    @pl.when(p < NSB)
    def _gather():
        b = p // NS
        s = p % NS
        base = s * ST
        hl = hl_ref[0]
        ub = pl.multiple_of(b * U, U)
        mb = pl.multiple_of(b * M, M)
        ucb = uc_ref[pl.ds(ub, U)]
        gu = (jax.lax.broadcasted_iota(_I32, (U, ST), 1) + base
              == ucb).astype(_F32)
        hu_s[pl.ds(ub, U)] += jnp.dot(
            gu, hl, preferred_element_type=_F32).astype(_BF16)
        mcb = mc_ref[pl.ds(mb, M)]
        gm = (jax.lax.broadcasted_iota(_I32, (M, ST), 1) + base
              == mcb).astype(_F32)
        hm_s[pl.ds(mb, M)] += jnp.dot(gm, hl, preferred_element_type=_F32)

    @pl.when(p == NSB)
    def _init():
        logits = jnp.dot(hm_s[...], wr_ref[...], preferred_element_type=_F32)
        logits = logits + br_ref[...]
        mx = jnp.max(logits, axis=-1, keepdims=True)
        e = jnp.exp(logits - mx)
        w_s[...] = e / jnp.sum(e, axis=-1, keepdims=True)
        diff = jnp.abs(ur_ref[...] - mc_ref[...])
        same_b = (jax.lax.broadcasted_iota(_I32, (BM, BU), 0) // M
                  == jax.lax.broadcasted_iota(_I32, (BM, BU), 1) // U)
        adj = ((diff > 0) & (diff <= r_ref[0]) & same_b).astype(_F32)
        cnt = jnp.sum(adj, axis=-1, keepdims=True)
        adjn_s[...] = adj / jnp.maximum(cnt, 1.0)
        cpos_s[...] = (cnt > 0.0).astype(_F32)
        mix_s[...] = jnp.dot(w_s[...], b2_ref[...], preferred_element_type=_F32)

    @pl.when((p >= NSB) & (p < NSB + K))
    def _expert():
        k = p - NSB
        hid = jnp.dot(hu_s[...], w1_ref[0].astype(_BF16),
                      preferred_element_type=_F32)
        hid = _gelu_exact(hid + b1_ref[0]).astype(_BF16)
        sel = (jax.lax.broadcasted_iota(_I32, (1, K), 1) == k).astype(_F32)
        w_col = jnp.sum(w_s[...] * sel, axis=-1, keepdims=True)
        aw = (adjn_s[...] * w_col).astype(_BF16)
        t = jnp.dot(aw, hid, preferred_element_type=_F32).astype(_BF16)
        mix_s[...] += jnp.dot(t, w2_ref[0].astype(_BF16),
                              preferred_element_type=_F32)

    @pl.when(p == NSB + K - 1)
    def _fin():
        mix = mix_s[...]
        mu = jnp.mean(mix, axis=-1, keepdims=True)
        var = jnp.mean((mix - mu) ** 2, axis=-1, keepdims=True)
        mix_s[...] = (mix - mu) * jax.lax.rsqrt(var + 1e-5) * cpos_s[...]

    @pl.when(p >= NSB + K)
    def _scatter():
        q = p - NSB - K
        b = q // NS
        s = q % NS
        base = s * ST
        mr = mr_ref[0]
        nxt = jnp.concatenate([mr[:, 1:], jnp.full((1, 1), -1, _I32)], axis=1)
        last = mr != nxt
        col = jax.lax.broadcasted_iota(_I32, (ST, M), 0) + base
        pm = ((col == mr) & last).astype(_F32)
        mb = pl.multiple_of(b * M, M)
        out_ref[0] = jnp.dot(pm, mix_s[pl.ds(mb, M)],
                             preferred_element_type=_F32)


def kernel(h_L, mask_indices, unmasked_indices, range_r, W_r, b_r,
           W1, b1, W2, b2):
    B, S, D = h_L.shape
    M = mask_indices.shape[1]
    U = unmasked_indices.shape[1]
    K = W_r.shape[1]
    F = W1.shape[2]
    ST = 512
    NS = S // ST
    NSB = B * NS
    mi = mask_indices.astype(_I32)
    ui = unmasked_indices.astype(_I32)
    r_arr = jnp.asarray(range_r, _I32).reshape(1)
    grid = (NSB + K + NSB,)

    def hl_idx(p):
        b = jnp.where(p < NSB, p // NS, B - 1)
        s = jnp.where(p < NSB, p % NS, NS - 1)
        return (b, s, 0)

    def w_idx(p):
        return (jnp.clip(p - NSB, 0, K - 1), 0, 0)

    def mr_idx(p):
        return (jnp.clip((p - NSB - K) // NS, 0, B - 1), 0, 0)

    def out_idx(p):
        q = jnp.clip(p - NSB - K, 0, NSB - 1)
        return (q // NS, q % NS, 0)

    out = pl.pallas_call(
        functools.partial(_body, B=B, NS=NS, ST=ST, U=U, M=M, K=K, D=D),
        grid=grid,
        in_specs=[
            pl.BlockSpec((1, ST, D), hl_idx),
            pl.BlockSpec((B * M, 1), lambda p: (0, 0)),
            pl.BlockSpec((1, 1, M), mr_idx),
            pl.BlockSpec((B * U, 1), lambda p: (0, 0)),
            pl.BlockSpec((1, B * U), lambda p: (0, 0)),
            pl.BlockSpec(memory_space=pltpu.SMEM),
            pl.BlockSpec((D, K), lambda p: (0, 0)),
            pl.BlockSpec((1, K), lambda p: (0, 0)),
            pl.BlockSpec((1, D, F), w_idx),
            pl.BlockSpec((1, 1, F), w_idx),
            pl.BlockSpec((1, F, D), w_idx),
            pl.BlockSpec((K, D), lambda p: (0, 0)),
        ],
        out_specs=pl.BlockSpec((1, ST, D), out_idx),
        out_shape=jax.ShapeDtypeStruct((B, S, D), _F32),
        scratch_shapes=[
            pltpu.VMEM((B * U, D), _BF16),
            pltpu.VMEM((B * M, D), _F32),
            pltpu.VMEM((B * M, K), _F32),
            pltpu.VMEM((B * M, B * U), _F32),
            pltpu.VMEM((B * M, 1), _F32),
            pltpu.VMEM((B * M, D), _F32),
        ],
    )(h_L, mi.reshape(B * M, 1), mi.reshape(B, 1, M),
      ui.reshape(B * U, 1), ui.reshape(1, B * U), r_arr,
      W_r, b_r.reshape(1, K), W1, b1.reshape(K, 1, F), W2, b2)
    return out
